# Initial kernel scaffold; baseline (speedup 1.0000x reference)
import jax
import jax.numpy as jnp
from jax.experimental import pallas as pl


def kernel(traj_loc, mat2, vec, traj_len, emb_su, emb_sl, emb_tu, emb_tl):
    raise NotImplementedError("write your pallas kernel here")


# speedup vs baseline: 10.4709x; 10.4709x over previous
import jax
import jax.numpy as jnp
from jax.experimental import pallas as pl
from jax.experimental.pallas import tpu as pltpu

_B, _L, _LOC_MAX, _EMB = 4, 50, 2000, 16
_SU, _SL, _TU, _TL = 100.0, 0.0, 500.0, 0.0


def _body(idx_ref, vf_ref, vecv_ref, esl_ref, esu_ref, etl_ref, etu_ref,
          row_ref, out_ref):
    p = pl.program_id(0)
    v = vf_ref[p]
    t = vecv_ref[p]

    def sel(ref):
        lo = ref[0:1, :]
        return lo + v * (ref[1:2, :] - lo)

    esl = sel(esl_ref)
    esu = sel(esu_ref)
    etl = sel(etl_ref)
    etu = sel(etu_ref)
    base = esl + etl + (etu - etl) * (t * (1.0 / _TU))
    coef = (esu - esl) * (v * (1.0 / _SU))

    row = row_ref[0]
    outer = jax.lax.dot_general(
        row, coef, (((0,), (0,)), ((), ())),
        preferred_element_type=jnp.float32)
    out_ref[0, 0] = outer + base


def kernel(traj_loc, mat2, vec, traj_len, emb_su, emb_sl, emb_tu, emb_tl):
    idx = (traj_loc.reshape(-1) - 1).astype(jnp.int32)
    vf = (jnp.arange(_L)[None, :] < traj_len[:, None]).astype(
        jnp.float32).reshape(-1)
    vecv = vec.reshape(-1).astype(jnp.float32)

    grid_spec = pltpu.PrefetchScalarGridSpec(
        num_scalar_prefetch=3,
        grid=(_B * _L,),
        in_specs=[
            pl.BlockSpec((2, _EMB), lambda p, i, f, t: (0, 0)),
            pl.BlockSpec((2, _EMB), lambda p, i, f, t: (0, 0)),
            pl.BlockSpec((2, _EMB), lambda p, i, f, t: (0, 0)),
            pl.BlockSpec((2, _EMB), lambda p, i, f, t: (0, 0)),
            pl.BlockSpec((1, 1, _LOC_MAX), lambda p, i, f, t: (i[p], 0, 0)),
        ],
        out_specs=pl.BlockSpec(
            (1, 1, _LOC_MAX, _EMB),
            lambda p, i, f, t: (p // _L, p % _L, 0, 0)),
    )
    out = pl.pallas_call(
        _body,
        grid_spec=grid_spec,
        out_shape=jax.ShapeDtypeStruct((_B, _L, _LOC_MAX, _EMB), jnp.float32),
    )(idx, vf, vecv, emb_sl, emb_su, emb_tl, emb_tu,
      mat2.reshape(_LOC_MAX, 1, _LOC_MAX))
    return out
